# baseline (speedup 1.0000x reference)
import jax
import jax.numpy as jnp
from jax.experimental import pallas as pl
from jax.experimental.pallas import tpu as pltpu

B = 4096
HID = 2048
LATENT_DIM = 256
NUM_EMBEDDINGS = 1024
COMMITMENT_COST = 0.25

MBLK = 1024
NM = B // MBLK
CCHUNK = 512
NC = HID // CCHUNK
NSTEP = 1 + 3 * NC + 1


def _fused_kernel(x_ref, w0_ref, b0_ref, w123_ref, b123_ref,
                  wlat_ref, blat_ref, emb_ref, embt_ref, esq_ref,
                  q_ref, idx_ref, loss_ref, perp_ref,
                  s0, s1, loss_acc, cnt_acc):
    m = pl.program_id(0)
    s = pl.program_id(1)

    @pl.when(s == 0)
    def _input_layer():
        h = jnp.dot(x_ref[...], w0_ref[...],
                    preferred_element_type=jnp.float32)
        s0[...] = jnp.maximum(h + b0_ref[...], 0.0)

    c = jax.lax.rem(jnp.clip(s - 1, 0, 3 * NC - 1), NC)
    col = pl.ds(c * CCHUNK, CCHUNK)

    @pl.when(jnp.logical_or(jnp.logical_and(s >= 1, s <= NC),
                            jnp.logical_and(s >= 1 + 2 * NC, s <= 3 * NC)))
    def _hidden_even():
        h = jnp.dot(s0[...], w123_ref[0], preferred_element_type=jnp.float32)
        s1[:, col] = jnp.maximum(h + b123_ref[0, 0], 0.0)

    @pl.when(jnp.logical_and(s >= 1 + NC, s <= 2 * NC))
    def _hidden_odd():
        h = jnp.dot(s1[...], w123_ref[0], preferred_element_type=jnp.float32)
        s0[:, col] = jnp.maximum(h + b123_ref[0, 0], 0.0)

    @pl.when(s == NSTEP - 1)
    def _vq_tail():
        @pl.when(m == 0)
        def _init():
            loss_acc[0, 0] = 0.0
            cnt_acc[...] = jnp.zeros_like(cnt_acc)

        z = jnp.dot(s1[...], wlat_ref[...],
                    preferred_element_type=jnp.float32) + blat_ref[...]
        zsq = jnp.sum(z * z, axis=1, keepdims=True)
        d = zsq - 2.0 * jnp.dot(z, emb_ref[...],
                                preferred_element_type=jnp.float32) + esq_ref[...]
        dmin = jnp.min(d, axis=1, keepdims=True)
        lane = jax.lax.broadcasted_iota(jnp.int32, d.shape, 1)
        idx = jnp.min(jnp.where(d == dmin, lane, NUM_EMBEDDINGS), axis=1)
        onehot = (lane == idx[:, None]).astype(jnp.float32)
        q = jnp.dot(onehot, embt_ref[...], preferred_element_type=jnp.float32)
        q_ref[...] = q
        idx_ref[...] = idx[None, :]

        diff = q - z
        loss_acc[0, 0] += jnp.sum(diff * diff)
        cnt_acc[...] += jnp.sum(onehot, axis=0)[None, :]

        @pl.when(m == NM - 1)
        def _fini():
            loss_ref[...] = jnp.reshape(
                (COMMITMENT_COST / (B * LATENT_DIM)) * loss_acc[0, 0], (1, 1))
            p = cnt_acc[...] * (1.0 / B)
            ent = jnp.sum(p * jnp.log(p + 1e-10))
            perp_ref[...] = jnp.reshape(jnp.exp(-ent), (1, 1))


def _w123_map(m, s):
    sc = jnp.clip(s - 1, 0, 3 * NC - 1)
    return (sc // NC, 0, jax.lax.rem(sc, NC))


def kernel(obs, action, next_obs, reward, termination,
           W0, b0, W1, b1, W2, b2, W3, b3, W_lat, b_lat,
           embeddings, is_training):
    x = jnp.hstack([obs, action, next_obs, reward, termination])
    in_dim = x.shape[1]
    W123 = jnp.stack([W1, W2, W3])
    b123 = jnp.stack([b1, b2, b3]).reshape(3, 1, HID)
    esq = jnp.sum(embeddings * embeddings, axis=0, keepdims=True)

    q, idx, loss, perp = pl.pallas_call(
        _fused_kernel,
        grid=(NM, NSTEP),
        in_specs=[
            pl.BlockSpec((MBLK, in_dim), lambda m, s: (m, 0)),
            pl.BlockSpec((in_dim, HID), lambda m, s: (0, 0)),
            pl.BlockSpec((1, HID), lambda m, s: (0, 0)),
            pl.BlockSpec((1, HID, CCHUNK), _w123_map),
            pl.BlockSpec((1, 1, CCHUNK),
                         lambda m, s: (_w123_map(m, s)[0], 0,
                                       _w123_map(m, s)[2])),
            pl.BlockSpec((HID, LATENT_DIM), lambda m, s: (0, 0)),
            pl.BlockSpec((1, LATENT_DIM), lambda m, s: (0, 0)),
            pl.BlockSpec((LATENT_DIM, NUM_EMBEDDINGS), lambda m, s: (0, 0)),
            pl.BlockSpec((NUM_EMBEDDINGS, LATENT_DIM), lambda m, s: (0, 0)),
            pl.BlockSpec((1, NUM_EMBEDDINGS), lambda m, s: (0, 0)),
        ],
        out_specs=[
            pl.BlockSpec((MBLK, LATENT_DIM), lambda m, s: (m, 0)),
            pl.BlockSpec((1, MBLK), lambda m, s: (0, m)),
            pl.BlockSpec((1, 1), lambda m, s: (0, 0)),
            pl.BlockSpec((1, 1), lambda m, s: (0, 0)),
        ],
        out_shape=[
            jax.ShapeDtypeStruct((B, LATENT_DIM), jnp.float32),
            jax.ShapeDtypeStruct((1, B), jnp.int32),
            jax.ShapeDtypeStruct((1, 1), jnp.float32),
            jax.ShapeDtypeStruct((1, 1), jnp.float32),
        ],
        scratch_shapes=[
            pltpu.VMEM((MBLK, HID), jnp.float32),
            pltpu.VMEM((MBLK, HID), jnp.float32),
            pltpu.SMEM((1, 1), jnp.float32),
            pltpu.VMEM((1, NUM_EMBEDDINGS), jnp.float32),
        ],
    )(x, W0, b0.reshape(1, HID), W123, b123,
      W_lat, b_lat.reshape(1, LATENT_DIM), embeddings, embeddings.T, esq)
    return q, loss.reshape(()), perp.reshape(()), idx.reshape(B)


# speedup vs baseline: 1.0380x; 1.0380x over previous
import jax
import jax.numpy as jnp
from jax.experimental import pallas as pl
from jax.experimental.pallas import tpu as pltpu

B = 4096
HID = 2048
LATENT_DIM = 256
NUM_EMBEDDINGS = 1024
COMMITMENT_COST = 0.25

MBLK = 1024
NM = B // MBLK
CCHUNK = 1024
NC = HID // CCHUNK
NSTEP = 1 + 3 * NC + 1


def _fused_kernel(x_ref, w0_ref, b0_ref, w123_ref, b123_ref,
                  wlat_ref, blat_ref, emb_ref, embt_ref, esq_ref,
                  q_ref, idx_ref, loss_ref, perp_ref,
                  s0, s1, loss_acc, cnt_acc):
    m = pl.program_id(0)
    s = pl.program_id(1)

    @pl.when(s == 0)
    def _input_layer():
        h = jnp.dot(x_ref[...], w0_ref[...],
                    preferred_element_type=jnp.float32)
        s0[...] = jnp.maximum(h + b0_ref[...], 0.0)

    c = jax.lax.rem(jnp.clip(s - 1, 0, 3 * NC - 1), NC)
    col = pl.ds(c * CCHUNK, CCHUNK)

    @pl.when(jnp.logical_or(jnp.logical_and(s >= 1, s <= NC),
                            jnp.logical_and(s >= 1 + 2 * NC, s <= 3 * NC)))
    def _hidden_even():
        h = jnp.dot(s0[...], w123_ref[0], preferred_element_type=jnp.float32)
        s1[:, col] = jnp.maximum(h + b123_ref[0, 0], 0.0)

    @pl.when(jnp.logical_and(s >= 1 + NC, s <= 2 * NC))
    def _hidden_odd():
        h = jnp.dot(s1[...], w123_ref[0], preferred_element_type=jnp.float32)
        s0[:, col] = jnp.maximum(h + b123_ref[0, 0], 0.0)

    @pl.when(s == NSTEP - 1)
    def _vq_tail():
        @pl.when(m == 0)
        def _init():
            loss_acc[0, 0] = 0.0
            cnt_acc[...] = jnp.zeros_like(cnt_acc)

        z = jnp.dot(s1[...], wlat_ref[...],
                    preferred_element_type=jnp.float32) + blat_ref[...]
        zsq = jnp.sum(z * z, axis=1, keepdims=True)
        d = zsq - 2.0 * jnp.dot(z, emb_ref[...],
                                preferred_element_type=jnp.float32) + esq_ref[...]
        dmin = jnp.min(d, axis=1, keepdims=True)
        lane = jax.lax.broadcasted_iota(jnp.int32, d.shape, 1)
        idx = jnp.min(jnp.where(d == dmin, lane, NUM_EMBEDDINGS), axis=1)
        onehot = (lane == idx[:, None]).astype(jnp.float32)
        q = jnp.dot(onehot, embt_ref[...], preferred_element_type=jnp.float32)
        q_ref[...] = q
        idx_ref[...] = idx[None, :]

        diff = q - z
        loss_acc[0, 0] += jnp.sum(diff * diff)
        cnt_acc[...] += jnp.sum(onehot, axis=0)[None, :]

        @pl.when(m == NM - 1)
        def _fini():
            loss_ref[...] = jnp.reshape(
                (COMMITMENT_COST / (B * LATENT_DIM)) * loss_acc[0, 0], (1, 1))
            p = cnt_acc[...] * (1.0 / B)
            ent = jnp.sum(p * jnp.log(p + 1e-10))
            perp_ref[...] = jnp.reshape(jnp.exp(-ent), (1, 1))


def _w123_map(m, s):
    sc = jnp.clip(s - 1, 0, 3 * NC - 1)
    return (sc // NC, 0, jax.lax.rem(sc, NC))


def kernel(obs, action, next_obs, reward, termination,
           W0, b0, W1, b1, W2, b2, W3, b3, W_lat, b_lat,
           embeddings, is_training):
    x = jnp.hstack([obs, action, next_obs, reward, termination])
    in_dim = x.shape[1]
    W123 = jnp.stack([W1, W2, W3])
    b123 = jnp.stack([b1, b2, b3]).reshape(3, 1, HID)
    esq = jnp.sum(embeddings * embeddings, axis=0, keepdims=True)

    q, idx, loss, perp = pl.pallas_call(
        _fused_kernel,
        grid=(NM, NSTEP),
        in_specs=[
            pl.BlockSpec((MBLK, in_dim), lambda m, s: (m, 0)),
            pl.BlockSpec((in_dim, HID), lambda m, s: (0, 0)),
            pl.BlockSpec((1, HID), lambda m, s: (0, 0)),
            pl.BlockSpec((1, HID, CCHUNK), _w123_map),
            pl.BlockSpec((1, 1, CCHUNK),
                         lambda m, s: (_w123_map(m, s)[0], 0,
                                       _w123_map(m, s)[2])),
            pl.BlockSpec((HID, LATENT_DIM), lambda m, s: (0, 0)),
            pl.BlockSpec((1, LATENT_DIM), lambda m, s: (0, 0)),
            pl.BlockSpec((LATENT_DIM, NUM_EMBEDDINGS), lambda m, s: (0, 0)),
            pl.BlockSpec((NUM_EMBEDDINGS, LATENT_DIM), lambda m, s: (0, 0)),
            pl.BlockSpec((1, NUM_EMBEDDINGS), lambda m, s: (0, 0)),
        ],
        out_specs=[
            pl.BlockSpec((MBLK, LATENT_DIM), lambda m, s: (m, 0)),
            pl.BlockSpec((1, MBLK), lambda m, s: (0, m)),
            pl.BlockSpec((1, 1), lambda m, s: (0, 0)),
            pl.BlockSpec((1, 1), lambda m, s: (0, 0)),
        ],
        out_shape=[
            jax.ShapeDtypeStruct((B, LATENT_DIM), jnp.float32),
            jax.ShapeDtypeStruct((1, B), jnp.int32),
            jax.ShapeDtypeStruct((1, 1), jnp.float32),
            jax.ShapeDtypeStruct((1, 1), jnp.float32),
        ],
        scratch_shapes=[
            pltpu.VMEM((MBLK, HID), jnp.float32),
            pltpu.VMEM((MBLK, HID), jnp.float32),
            pltpu.SMEM((1, 1), jnp.float32),
            pltpu.VMEM((1, NUM_EMBEDDINGS), jnp.float32),
        ],
    )(x, W0, b0.reshape(1, HID), W123, b123,
      W_lat, b_lat.reshape(1, LATENT_DIM), embeddings, embeddings.T, esq)
    return q, loss.reshape(()), perp.reshape(()), idx.reshape(B)


# speedup vs baseline: 1.1859x; 1.1424x over previous
import jax
import jax.numpy as jnp
from jax.experimental import pallas as pl
from jax.experimental.pallas import tpu as pltpu

B = 4096
OBS_DIM = 256
ACT_DIM = 64
HID = 2048
LATENT_DIM = 256
NUM_EMBEDDINGS = 1024
COMMITMENT_COST = 0.25

MBLK = 1024
NM = B // MBLK
CCHUNK = 512
NC = HID // CCHUNK
NH = 3 * NC
NSLOT = 4
NSTEP = 1 + NH + 1


def _fused_kernel(x_ref, w0_ref, b0_ref,
                  w1_ref, w2_ref, w3_ref, b1_ref, b2_ref, b3_ref,
                  wlat_ref, blat_ref, emb_ref, embt_ref, esq_ref,
                  q_ref, idx_ref, loss_ref, perp_ref,
                  s0, s1, wbuf, sems, loss_acc, cnt_acc):
    m = pl.program_id(0)
    s = pl.program_id(1)
    whbm = (w1_ref, w2_ref, w3_ref)

    def copy(j, slot):
        src = whbm[j // NC]
        c = j % NC
        return pltpu.make_async_copy(
            src.at[:, pl.ds(c * CCHUNK, CCHUNK)], wbuf.at[slot],
            sems.at[slot])

    @pl.when(jnp.logical_and(s == 0, m == 0))
    def _first_issue():
        copy(0, 0).start()
        copy(1, 1).start()

    @pl.when(s == 0)
    def _input_layer():
        h = jnp.dot(x_ref[...], w0_ref[...],
                    preferred_element_type=jnp.float32)
        s0[...] = jnp.maximum(h + b0_ref[...], 0.0)

    bias = (b1_ref, b2_ref, b3_ref)
    scr = (s0, s1)
    for j in range(NH):
        @pl.when(s == 1 + j)
        def _hidden(j=j):
            slot = j % NSLOT
            if j + 2 < NH:
                copy(j + 2, (j + 2) % NSLOT).start()
            copy(j, slot).wait()
            layer = j // NC
            c = j % NC
            src = scr[layer % 2]
            dst = scr[(layer + 1) % 2]
            h = jnp.dot(src[...], wbuf[slot],
                        preferred_element_type=jnp.float32)
            dst[:, c * CCHUNK:(c + 1) * CCHUNK] = jnp.maximum(
                h + bias[layer][:, c * CCHUNK:(c + 1) * CCHUNK], 0.0)

    @pl.when(s == NSTEP - 1)
    def _vq_tail():
        @pl.when(m < NM - 1)
        def _issue_next():
            copy(0, 0).start()
            copy(1, 1).start()

        @pl.when(m == 0)
        def _init():
            loss_acc[0, 0] = 0.0
            cnt_acc[...] = jnp.zeros_like(cnt_acc)

        z = jnp.dot(s1[...], wlat_ref[...],
                    preferred_element_type=jnp.float32) + blat_ref[...]
        zsq = jnp.sum(z * z, axis=1, keepdims=True)
        d = zsq - 2.0 * jnp.dot(z, emb_ref[...],
                                preferred_element_type=jnp.float32) + esq_ref[...]
        dmin = jnp.min(d, axis=1, keepdims=True)
        lane = jax.lax.broadcasted_iota(jnp.int32, d.shape, 1)
        idx = jnp.min(jnp.where(d == dmin, lane, NUM_EMBEDDINGS), axis=1)
        onehot = (lane == idx[:, None]).astype(jnp.float32)
        q = jnp.dot(onehot, embt_ref[...], preferred_element_type=jnp.float32)
        q_ref[...] = q
        idx_ref[...] = idx[None, :]

        diff = q - z
        loss_acc[0, 0] += jnp.sum(diff * diff)
        cnt_acc[...] += jnp.sum(onehot, axis=0)[None, :]

        @pl.when(m == NM - 1)
        def _fini():
            loss_ref[...] = jnp.reshape(
                (COMMITMENT_COST / (B * LATENT_DIM)) * loss_acc[0, 0], (1, 1))
            p = cnt_acc[...] * (1.0 / B)
            ent = jnp.sum(p * jnp.log(p + 1e-10))
            perp_ref[...] = jnp.reshape(jnp.exp(-ent), (1, 1))


def kernel(obs, action, next_obs, reward, termination,
           W0, b0, W1, b1, W2, b2, W3, b3, W_lat, b_lat,
           embeddings, is_training):
    x = jnp.hstack([obs, action, next_obs, reward, termination])
    in_dim = x.shape[1]
    esq = jnp.sum(embeddings ** 2, axis=0, keepdims=True)

    rows = lambda m, s: (m, 0)
    const = lambda m, s: (0, 0)
    hbm = pl.BlockSpec(memory_space=pl.ANY)

    q, idx, loss, perp = pl.pallas_call(
        _fused_kernel,
        grid=(NM, NSTEP),
        in_specs=[
            pl.BlockSpec((MBLK, in_dim), rows),
            pl.BlockSpec((in_dim, HID), const),
            pl.BlockSpec((1, HID), const),
            hbm, hbm, hbm,
            pl.BlockSpec((1, HID), const),
            pl.BlockSpec((1, HID), const),
            pl.BlockSpec((1, HID), const),
            pl.BlockSpec((HID, LATENT_DIM), const),
            pl.BlockSpec((1, LATENT_DIM), const),
            pl.BlockSpec((LATENT_DIM, NUM_EMBEDDINGS), const),
            pl.BlockSpec((NUM_EMBEDDINGS, LATENT_DIM), const),
            pl.BlockSpec((1, NUM_EMBEDDINGS), const),
        ],
        out_specs=[
            pl.BlockSpec((MBLK, LATENT_DIM), rows),
            pl.BlockSpec((1, MBLK), lambda m, s: (0, m)),
            pl.BlockSpec((1, 1), const),
            pl.BlockSpec((1, 1), const),
        ],
        out_shape=[
            jax.ShapeDtypeStruct((B, LATENT_DIM), jnp.float32),
            jax.ShapeDtypeStruct((1, B), jnp.int32),
            jax.ShapeDtypeStruct((1, 1), jnp.float32),
            jax.ShapeDtypeStruct((1, 1), jnp.float32),
        ],
        scratch_shapes=[
            pltpu.VMEM((MBLK, HID), jnp.float32),
            pltpu.VMEM((MBLK, HID), jnp.float32),
            pltpu.VMEM((NSLOT, HID, CCHUNK), jnp.float32),
            pltpu.SemaphoreType.DMA((NSLOT,)),
            pltpu.SMEM((1, 1), jnp.float32),
            pltpu.VMEM((1, NUM_EMBEDDINGS), jnp.float32),
        ],
    )(x, W0, b0.reshape(1, HID),
      W1, W2, W3,
      b1.reshape(1, HID), b2.reshape(1, HID), b3.reshape(1, HID),
      W_lat, b_lat.reshape(1, LATENT_DIM), embeddings, embeddings.T, esq)
    return q, loss.reshape(()), perp.reshape(()), idx.reshape(B)


# speedup vs baseline: 1.2674x; 1.0687x over previous
import jax
import jax.numpy as jnp
from jax.experimental import pallas as pl
from jax.experimental.pallas import tpu as pltpu

B = 4096
OBS_DIM = 256
IN_DIM = 578
ACT_DIM = 64
HID = 2048
LATENT_DIM = 256
NUM_EMBEDDINGS = 1024
COMMITMENT_COST = 0.25

MBLK = 1024
NM = B // MBLK
CCHUNK = 512
NC = HID // CCHUNK
NH = 3 * NC
NSLOT = 4
NSTEP = 1 + NH + 1


def _fused_kernel(obs_ref, act_ref, nobs_ref, rew_ref, term_ref,
                  w0_ref, b0_ref,
                  w1_ref, w2_ref, w3_ref, b1_ref, b2_ref, b3_ref,
                  wlat_ref, blat_ref, emb_ref, esq_ref,
                  q_ref, idx_ref, loss_ref, perp_ref,
                  xbuf, s0, s1, wbuf, sems, loss_acc, cnt_acc):
    m = pl.program_id(0)
    s = pl.program_id(1)
    whbm = (w1_ref, w2_ref, w3_ref)

    def copy(j, slot):
        src = whbm[j // NC]
        c = j % NC
        return pltpu.make_async_copy(
            src.at[:, pl.ds(c * CCHUNK, CCHUNK)], wbuf.at[slot],
            sems.at[slot])

    @pl.when(jnp.logical_and(s == 0, m == 0))
    def _first_issue():
        copy(0, 0).start()
        copy(1, 1).start()

    @pl.when(s == 0)
    def _input_layer():
        xbuf[:, 0:OBS_DIM] = obs_ref[...]
        xbuf[:, OBS_DIM:OBS_DIM + ACT_DIM] = act_ref[...]
        xbuf[:, OBS_DIM + ACT_DIM:2 * OBS_DIM + ACT_DIM] = nobs_ref[...]
        xbuf[:, 2 * OBS_DIM + ACT_DIM:2 * OBS_DIM + ACT_DIM + 1] = rew_ref[...]
        xbuf[:, 2 * OBS_DIM + ACT_DIM + 1:IN_DIM] = term_ref[...]
        h = jnp.dot(xbuf[...], w0_ref[...],
                    preferred_element_type=jnp.float32)
        s0[...] = jnp.maximum(h + b0_ref[...], 0.0)

    bias = (b1_ref, b2_ref, b3_ref)
    scr = (s0, s1)
    for j in range(NH):
        @pl.when(s == 1 + j)
        def _hidden(j=j):
            slot = j % NSLOT
            if j + 2 < NH:
                copy(j + 2, (j + 2) % NSLOT).start()
            copy(j, slot).wait()
            layer = j // NC
            c = j % NC
            src = scr[layer % 2]
            dst = scr[(layer + 1) % 2]
            h = jnp.dot(src[...], wbuf[slot],
                        preferred_element_type=jnp.float32)
            dst[:, c * CCHUNK:(c + 1) * CCHUNK] = jnp.maximum(
                h + bias[layer][:, c * CCHUNK:(c + 1) * CCHUNK], 0.0)

    @pl.when(s == NSTEP - 1)
    def _vq_tail():
        @pl.when(m < NM - 1)
        def _issue_next():
            copy(0, 0).start()
            copy(1, 1).start()

        @pl.when(m == 0)
        def _init():
            loss_acc[0, 0] = 0.0
            cnt_acc[...] = jnp.zeros_like(cnt_acc)

        z = jnp.dot(s1[...], wlat_ref[...],
                    preferred_element_type=jnp.float32) + blat_ref[...]
        zsq = jnp.sum(z * z, axis=1, keepdims=True)
        d = zsq - 2.0 * jnp.dot(z, emb_ref[...],
                                preferred_element_type=jnp.float32) + esq_ref[...]
        dmin = jnp.min(d, axis=1, keepdims=True)
        lane = jax.lax.broadcasted_iota(jnp.int32, d.shape, 1)
        idx = jnp.min(jnp.where(d == dmin, lane, NUM_EMBEDDINGS), axis=1)
        onehot = (lane == idx[:, None]).astype(jnp.float32)
        q = jax.lax.dot_general(
            onehot, emb_ref[...], (((1,), (1,)), ((), ())),
            preferred_element_type=jnp.float32)
        q_ref[...] = q
        idx_ref[...] = idx[None, :]

        diff = q - z
        loss_acc[0, 0] += jnp.sum(diff * diff)
        cnt_acc[...] += jnp.sum(onehot, axis=0)[None, :]

        @pl.when(m == NM - 1)
        def _fini():
            loss_ref[...] = jnp.reshape(
                (COMMITMENT_COST / (B * LATENT_DIM)) * loss_acc[0, 0], (1, 1))
            p = cnt_acc[...] * (1.0 / B)
            ent = jnp.sum(p * jnp.log(p + 1e-10))
            perp_ref[...] = jnp.reshape(jnp.exp(-ent), (1, 1))


def kernel(obs, action, next_obs, reward, termination,
           W0, b0, W1, b1, W2, b2, W3, b3, W_lat, b_lat,
           embeddings, is_training):
    esq = jnp.sum(embeddings ** 2, axis=0, keepdims=True)

    rows = lambda m, s: (m, 0)
    const = lambda m, s: (0, 0)
    hbm = pl.BlockSpec(memory_space=pl.ANY)

    q, idx, loss, perp = pl.pallas_call(
        _fused_kernel,
        grid=(NM, NSTEP),
        in_specs=[
            pl.BlockSpec((MBLK, OBS_DIM), rows),
            pl.BlockSpec((MBLK, ACT_DIM), rows),
            pl.BlockSpec((MBLK, OBS_DIM), rows),
            pl.BlockSpec((MBLK, 1), rows),
            pl.BlockSpec((MBLK, 1), rows),
            pl.BlockSpec((IN_DIM, HID), const),
            pl.BlockSpec((1, HID), const),
            hbm, hbm, hbm,
            pl.BlockSpec((1, HID), const),
            pl.BlockSpec((1, HID), const),
            pl.BlockSpec((1, HID), const),
            pl.BlockSpec((HID, LATENT_DIM), const),
            pl.BlockSpec((1, LATENT_DIM), const),
            pl.BlockSpec((LATENT_DIM, NUM_EMBEDDINGS), const),
            pl.BlockSpec((1, NUM_EMBEDDINGS), const),
        ],
        out_specs=[
            pl.BlockSpec((MBLK, LATENT_DIM), rows),
            pl.BlockSpec((1, MBLK), lambda m, s: (0, m)),
            pl.BlockSpec((1, 1), const),
            pl.BlockSpec((1, 1), const),
        ],
        out_shape=[
            jax.ShapeDtypeStruct((B, LATENT_DIM), jnp.float32),
            jax.ShapeDtypeStruct((1, B), jnp.int32),
            jax.ShapeDtypeStruct((1, 1), jnp.float32),
            jax.ShapeDtypeStruct((1, 1), jnp.float32),
        ],
        scratch_shapes=[
            pltpu.VMEM((MBLK, IN_DIM), jnp.float32),
            pltpu.VMEM((MBLK, HID), jnp.float32),
            pltpu.VMEM((MBLK, HID), jnp.float32),
            pltpu.VMEM((NSLOT, HID, CCHUNK), jnp.float32),
            pltpu.SemaphoreType.DMA((NSLOT,)),
            pltpu.SMEM((1, 1), jnp.float32),
            pltpu.VMEM((1, NUM_EMBEDDINGS), jnp.float32),
        ],
    )(obs, action, next_obs, reward, termination,
      W0, b0.reshape(1, HID),
      W1, W2, W3,
      b1.reshape(1, HID), b2.reshape(1, HID), b3.reshape(1, HID),
      W_lat, b_lat.reshape(1, LATENT_DIM), embeddings, esq)
    return q, loss.reshape(()), perp.reshape(()), idx.reshape(B)
